# baseline (speedup 1.0000x reference)
import functools

import jax
import jax.numpy as jnp
from jax import lax
from jax.experimental import pallas as pl
from jax.experimental.pallas import tpu as pltpu
from jax.experimental.pallas import tpu_sc as plsc

NU = 5000
NN = 10000
E = 320000
D = 128
NL = 3

NC, NS, L = 2, 16, 16
P = 5120
NPT = P // NS
TP = 2 * P
EPT = (E // 2) // NS
CH = 128
NCHUNK = 2 * (-(-EPT // (2 * CH)))
EPTP = NCHUNK * CH

_MESH = plsc.VectorSubcoreMesh(
    core_axis_name="c", subcore_axis_name="s", num_cores=NC, num_subcores=NS
)


def _rsqrt16(x):
    y = 0.5 * (1.0 + x)
    for _ in range(20):
        y = 0.5 * (y + x / y)
    return 1.0 / y


def _scale_rows(buf, dbuf):
    @pl.loop(0, NPT)
    def _(n):
        dv = dbuf[pl.ds(n * L, L)]
        for k in range(D // L):
            sl = pl.ds(k * L, L)
            buf[n, sl] = buf[n, sl] * dv


@functools.partial(
    pl.kernel,
    out_type=(
        jax.ShapeDtypeStruct((TP, D), jnp.float32),
        jax.ShapeDtypeStruct((TP * L,), jnp.float32),
    ),
    mesh=_MESH,
    scratch_types=[
        pltpu.VMEM_SHARED((P, D), jnp.float32),
        pltpu.VMEM((NCHUNK, CH), jnp.int32),
        pltpu.VMEM((CH, D), jnp.float32),
        pltpu.VMEM((NPT, D), jnp.float32),
        pltpu.VMEM((NPT * L,), jnp.float32),
    ],
)
def _k_init(x0, rows, y0, dinv_out, deg_sh, ridx, ones, sbuf, dbuf):
    c = lax.axis_index("c")
    s = lax.axis_index("s")
    n0 = s * NPT

    @pl.loop(0, CH)
    def _(i):
        for k in range(D // L):
            ones[i, pl.ds(k * L, L)] = jnp.full((L,), 1.0, jnp.float32)

    @pl.loop(0, NPT)
    def _(n):
        for k in range(D // L):
            sbuf[n, pl.ds(k * L, L)] = jnp.zeros((L,), jnp.float32)

    pltpu.sync_copy(rows.at[c, s], ridx)
    pltpu.sync_copy(sbuf, deg_sh.at[pl.ds(n0, NPT)])
    plsc.subcore_barrier()

    @pl.loop(0, NCHUNK)
    def _(j):
        pltpu.sync_copy(ones, deg_sh.at[ridx.at[j]], add=True)

    plsc.subcore_barrier()

    pltpu.sync_copy(deg_sh.at[pl.ds(n0, NPT)], sbuf)

    @pl.loop(0, NPT)
    def _(n):
        dbuf[pl.ds(n * L, L)] = _rsqrt16(sbuf[n, pl.ds(0, L)] + 1e-07)

    pltpu.sync_copy(x0.at[pl.ds(c * P + n0, NPT)], sbuf)
    _scale_rows(sbuf, dbuf)
    pltpu.sync_copy(sbuf, y0.at[pl.ds(c * P + n0, NPT)])
    pltpu.sync_copy(dbuf, dinv_out.at[pl.ds((c * P + n0) * L, NPT * L)])


@functools.partial(
    pl.kernel,
    out_type=(
        jax.ShapeDtypeStruct((TP, D), jnp.float32),
        jax.ShapeDtypeStruct((TP, D), jnp.float32),
    ),
    mesh=_MESH,
    scratch_types=[
        pltpu.VMEM_SHARED((P, D), jnp.float32),
        pltpu.VMEM((NCHUNK, CH), jnp.int32),
        pltpu.VMEM((NCHUNK, CH), jnp.int32),
        pltpu.VMEM((CH, D), jnp.float32),
        pltpu.VMEM((NPT, D), jnp.float32),
        pltpu.VMEM((NPT * L,), jnp.float32),
        pltpu.SemaphoreType.DMA,
    ],
)
def _k_layer(y_in, rows, cols, dinv, x_out, y_out,
             s_sh, ridx, cidx, gbuf, sbuf, dbuf, sem):
    c = lax.axis_index("c")
    s = lax.axis_index("s")
    n0 = s * NPT

    @pl.loop(0, NPT)
    def _(n):
        for k in range(D // L):
            sbuf[n, pl.ds(k * L, L)] = jnp.zeros((L,), jnp.float32)

    pltpu.sync_copy(rows.at[c, s], ridx)
    pltpu.sync_copy(cols.at[c, s], cidx)
    pltpu.sync_copy(sbuf, s_sh.at[pl.ds(n0, NPT)])
    plsc.subcore_barrier()

    @pl.loop(0, NCHUNK)
    def _(j):
        pltpu.async_copy(y_in.at[cidx.at[j]], gbuf, sem).wait()
        pltpu.sync_copy(gbuf, s_sh.at[ridx.at[j]], add=True)

    plsc.subcore_barrier()

    pltpu.sync_copy(s_sh.at[pl.ds(n0, NPT)], sbuf)
    pltpu.sync_copy(dinv.at[pl.ds((c * P + n0) * L, NPT * L)], dbuf)
    _scale_rows(sbuf, dbuf)
    pltpu.sync_copy(sbuf, x_out.at[pl.ds(c * P + n0, NPT)])
    _scale_rows(sbuf, dbuf)
    pltpu.sync_copy(sbuf, y_out.at[pl.ds(c * P + n0, NPT)])


@functools.partial(
    pl.kernel,
    out_type=jax.ShapeDtypeStruct((TP, D), jnp.float32),
    mesh=_MESH,
    scratch_types=[
        pltpu.VMEM((NPT, D), jnp.float32),
        pltpu.VMEM((NPT, D), jnp.float32),
    ],
)
def _k_final(x0, x1, x2, x3, total, abuf, bbuf):
    c = lax.axis_index("c")
    s = lax.axis_index("s")
    base = (c * NS + s) * NPT

    pltpu.sync_copy(x0.at[pl.ds(base, NPT)], abuf)
    for xin in (x1, x2, x3):
        pltpu.sync_copy(xin.at[pl.ds(base, NPT)], bbuf)

        @pl.loop(0, NPT)
        def _(n):
            for k in range(D // L):
                sl = pl.ds(k * L, L)
                abuf[n, sl] = abuf[n, sl] + bbuf[n, sl]

    @pl.loop(0, NPT)
    def _(n):
        for k in range(D // L):
            sl = pl.ds(k * L, L)
            abuf[n, sl] = abuf[n, sl] * 0.25

    pltpu.sync_copy(abuf, total.at[pl.ds(base, NPT)])


def kernel(user_emb, item_emb, edge_index):
    row = edge_index[0].astype(jnp.int32)
    col = edge_index[1].astype(jnp.int32)
    H = E // 2

    half = (jnp.arange(E, dtype=jnp.int32) >= H).astype(jnp.int32)
    rloc = row - half * NU
    colp = col + (P - NU) * (col >= NU).astype(jnp.int32)

    def tile_pack(a, pad_value):
        a = a.reshape(NC, NS, EPT)
        a = jnp.pad(a, ((0, 0), (0, 0), (0, EPTP - EPT)),
                    constant_values=pad_value)
        return a.reshape(NC, NS, NCHUNK, CH)

    rows = tile_pack(rloc, NU)
    cols = tile_pack(colp, 0)

    x0 = jnp.zeros((TP, D), jnp.float32)
    x0 = x0.at[0:NU].set(user_emb).at[P:P + NU].set(item_emb)

    y0, dinv = _k_init(x0, rows)
    x1, y1 = _k_layer(y0, rows, cols, dinv)
    x2, y2 = _k_layer(y1, rows, cols, dinv)
    x3, _ = _k_layer(y2, rows, cols, dinv)
    total = _k_final(x0, x1, x2, x3)

    return (total[0:NU], total[P:P + NU])


# speedup vs baseline: 1.4189x; 1.4189x over previous
import functools

import jax
import jax.numpy as jnp
from jax import lax
from jax.experimental import pallas as pl
from jax.experimental.pallas import tpu as pltpu
from jax.experimental.pallas import tpu_sc as plsc

NU = 5000
NN = 10000
E = 320000
D = 128
NL = 3

NC, NS, L = 2, 16, 16
P = 5120
NPT = P // NS
TP = 2 * P
EPT = (E // 2) // NS
CH = 128
NCHUNK = -(-EPT // CH)
EPTP = NCHUNK * CH

_MESH = plsc.VectorSubcoreMesh(
    core_axis_name="c", subcore_axis_name="s", num_cores=NC, num_subcores=NS
)


def _rsqrt16(x):
    y = 0.5 * (1.0 + x)
    for _ in range(20):
        y = 0.5 * (y + x / y)
    return 1.0 / y


def _scale_rows(buf, dbuf):
    @pl.loop(0, NPT)
    def _(n):
        dv = dbuf[pl.ds(n * L, L)]
        for k in range(D // L):
            sl = pl.ds(k * L, L)
            buf[n, sl] = buf[n, sl] * dv


@functools.partial(
    pl.kernel,
    out_type=(
        jax.ShapeDtypeStruct((TP, D), jnp.float32),
        jax.ShapeDtypeStruct((TP * L,), jnp.float32),
    ),
    mesh=_MESH,
    scratch_types=[
        pltpu.VMEM_SHARED((P, D), jnp.float32),
        pltpu.VMEM((NCHUNK, CH), jnp.int32),
        pltpu.VMEM((CH, D), jnp.float32),
        pltpu.VMEM((NPT, D), jnp.float32),
        pltpu.VMEM((NPT * L,), jnp.float32),
    ],
)
def _k_init(x0, rows, y0, dinv_out, deg_sh, ridx, ones, sbuf, dbuf):
    c = lax.axis_index("c")
    s = lax.axis_index("s")
    n0 = s * NPT

    @pl.loop(0, CH)
    def _(i):
        for k in range(D // L):
            ones[i, pl.ds(k * L, L)] = jnp.full((L,), 1.0, jnp.float32)

    @pl.loop(0, NPT)
    def _(n):
        for k in range(D // L):
            sbuf[n, pl.ds(k * L, L)] = jnp.zeros((L,), jnp.float32)

    pltpu.sync_copy(rows.at[c, s], ridx)
    pltpu.sync_copy(sbuf, deg_sh.at[pl.ds(n0, NPT)])
    plsc.subcore_barrier()

    @pl.loop(0, NCHUNK)
    def _(j):
        pltpu.sync_copy(ones, deg_sh.at[ridx.at[j]], add=True)

    plsc.subcore_barrier()

    pltpu.sync_copy(deg_sh.at[pl.ds(n0, NPT)], sbuf)

    @pl.loop(0, NPT)
    def _(n):
        dbuf[pl.ds(n * L, L)] = _rsqrt16(sbuf[n, pl.ds(0, L)] + 1e-07)

    pltpu.sync_copy(x0.at[pl.ds(c * P + n0, NPT)], sbuf)
    _scale_rows(sbuf, dbuf)
    pltpu.sync_copy(sbuf, y0.at[pl.ds(c * P + n0, NPT)])
    pltpu.sync_copy(dbuf, dinv_out.at[pl.ds((c * P + n0) * L, NPT * L)])


@functools.partial(
    pl.kernel,
    out_type=(
        jax.ShapeDtypeStruct((TP, D), jnp.float32),
        jax.ShapeDtypeStruct((TP, D), jnp.float32),
    ),
    mesh=_MESH,
    scratch_types=[
        pltpu.VMEM_SHARED((P, D), jnp.float32),
        pltpu.VMEM((NCHUNK, CH), jnp.int32),
        pltpu.VMEM((NCHUNK, CH), jnp.int32),
        pltpu.VMEM((CH, D), jnp.float32),
        pltpu.VMEM((NPT, D), jnp.float32),
        pltpu.VMEM((NPT * L,), jnp.float32),
        pltpu.SemaphoreType.DMA,
    ],
)
def _k_layer(y_in, rows, cols, dinv, x_out, y_out,
             s_sh, ridx, cidx, gbuf, sbuf, dbuf, sem):
    c = lax.axis_index("c")
    s = lax.axis_index("s")
    n0 = s * NPT

    @pl.loop(0, NPT)
    def _(n):
        for k in range(D // L):
            sbuf[n, pl.ds(k * L, L)] = jnp.zeros((L,), jnp.float32)

    pltpu.sync_copy(rows.at[c, s], ridx)
    pltpu.sync_copy(cols.at[c, s], cidx)
    pltpu.sync_copy(sbuf, s_sh.at[pl.ds(n0, NPT)])
    plsc.subcore_barrier()

    @pl.loop(0, NCHUNK)
    def _(j):
        pltpu.async_copy(y_in.at[cidx.at[j]], gbuf, sem).wait()
        pltpu.sync_copy(gbuf, s_sh.at[ridx.at[j]], add=True)

    plsc.subcore_barrier()

    pltpu.sync_copy(s_sh.at[pl.ds(n0, NPT)], sbuf)
    pltpu.sync_copy(dinv.at[pl.ds((c * P + n0) * L, NPT * L)], dbuf)
    _scale_rows(sbuf, dbuf)
    pltpu.sync_copy(sbuf, x_out.at[pl.ds(c * P + n0, NPT)])
    _scale_rows(sbuf, dbuf)
    pltpu.sync_copy(sbuf, y_out.at[pl.ds(c * P + n0, NPT)])


@functools.partial(
    pl.kernel,
    out_type=jax.ShapeDtypeStruct((TP, D), jnp.float32),
    mesh=_MESH,
    scratch_types=[
        pltpu.VMEM((NPT, D), jnp.float32),
        pltpu.VMEM((NPT, D), jnp.float32),
    ],
)
def _k_final(x0, x1, x2, x3, total, abuf, bbuf):
    c = lax.axis_index("c")
    s = lax.axis_index("s")
    base = (c * NS + s) * NPT

    pltpu.sync_copy(x0.at[pl.ds(base, NPT)], abuf)
    for xin in (x1, x2, x3):
        pltpu.sync_copy(xin.at[pl.ds(base, NPT)], bbuf)

        @pl.loop(0, NPT)
        def _(n):
            for k in range(D // L):
                sl = pl.ds(k * L, L)
                abuf[n, sl] = abuf[n, sl] + bbuf[n, sl]

    @pl.loop(0, NPT)
    def _(n):
        for k in range(D // L):
            sl = pl.ds(k * L, L)
            abuf[n, sl] = abuf[n, sl] * 0.25

    pltpu.sync_copy(abuf, total.at[pl.ds(base, NPT)])


def kernel(user_emb, item_emb, edge_index):
    row = edge_index[0].astype(jnp.int32)
    col = edge_index[1].astype(jnp.int32)
    H = E // 2

    half = (jnp.arange(E, dtype=jnp.int32) >= H).astype(jnp.int32)
    rloc = row - half * NU
    colp = col + (P - NU) * (col >= NU).astype(jnp.int32)

    def tile_pack(a, pad_block):
        a = a.reshape(NC, NS, EPT)
        a = jnp.concatenate([a, pad_block], axis=2)
        return a.reshape(NC, NS, NCHUNK, CH)

    npad = EPTP - EPT
    spread = (P - NU) // NS
    sids = jnp.arange(NS, dtype=jnp.int32)[None, :, None]
    kpad = jnp.arange(npad, dtype=jnp.int32)[None, None, :]
    row_pad = jnp.broadcast_to(NU + sids * spread + kpad % spread,
                               (NC, NS, npad)).astype(jnp.int32)
    col_pad = jnp.zeros((NC, NS, npad), jnp.int32)

    rows = tile_pack(rloc, row_pad)
    cols = tile_pack(colp, col_pad)

    x0 = jnp.zeros((TP, D), jnp.float32)
    x0 = x0.at[0:NU].set(user_emb).at[P:P + NU].set(item_emb)

    y0, dinv = _k_init(x0, rows)
    x1, y1 = _k_layer(y0, rows, cols, dinv)
    x2, y2 = _k_layer(y1, rows, cols, dinv)
    x3, _ = _k_layer(y2, rows, cols, dinv)
    total = _k_final(x0, x1, x2, x3)

    return (total[0:NU], total[P:P + NU])
